# baseline (speedup 1.0000x reference)
import functools

import jax
import jax.numpy as jnp
import numpy as np
from jax import lax
from jax.experimental import pallas as pl
from jax.experimental.pallas import tpu as pltpu
from jax.experimental.pallas import tpu_sc as plsc

B, H, W, C = 4, 384, 384, 96
HW = H * W
N = B * HW
NW = 32
ROWS_PER_TILE = N // NW
K = 128
NCHUNK = ROWS_PER_TILE // K
TILES_PER_BATCH = NW // B
STEP = float(np.float32(2.0) / np.float32(W - 1))


def _interp_body(x_hbm, t_hbm, out_hbm, t_v,
                 ia0, ib0, ic0, id0, ia1, ib1, ic1, id1,
                 wa0, wb0, wc0, wd0, wa1, wb1, wc1, wd1,
                 ra0, rb0, rc0, rd0, ra1, rb1, rc1, rd1,
                 gsem0, gsem1, osem0, osem1):
    wid = lax.axis_index("s") * 2 + lax.axis_index("c")
    b = wid // TILES_PER_BATCH
    pltpu.sync_copy(t_hbm.at[b], t_v)
    vsplit = jnp.full((16,), 65537.0, jnp.float32)

    def bf16r(x):
        p = x * vsplit
        q = p - x
        return p - q

    t0 = bf16r(t_v[0, :])
    t1 = bf16r(t_v[1, :])
    t2 = bf16r(t_v[2, :])
    t3 = bf16r(t_v[3, :])
    t4 = bf16r(t_v[4, :])
    t5 = bf16r(t_v[5, :])
    jloc0 = (wid % TILES_PER_BATCH) * ROWS_PER_TILE
    row0 = wid * ROWS_PER_TILE
    bbase = b * HW
    lanes = lax.iota(jnp.int32, 16)

    izero = jnp.zeros((16,), jnp.int32)
    iwmax = jnp.full((16,), W - 1, jnp.int32)
    ihmax = jnp.full((16,), H - 1, jnp.int32)
    ione = jnp.full((16,), 1, jnp.int32)
    wsplat = jnp.full((16,), W, jnp.int32)
    c7 = jnp.full((16,), 7, jnp.int32)
    c17 = jnp.full((16,), 17, jnp.int32)
    cmag = jnp.full((16,), 43691, jnp.int32)
    bvec = jnp.full((16,), bbase, jnp.int32)

    idx_sets = ((ia0, ib0, ic0, id0), (ia1, ib1, ic1, id1))
    w_sets = ((wa0, wb0, wc0, wd0), (wa1, wb1, wc1, wd1))
    row_sets = ((ra0, rb0, rc0, rd0), (ra1, rb1, rc1, rd1))
    out_sets = (ra0, ra1)
    gsems = (gsem0, gsem1)
    osems = (osem0, osem1)

    def fill_idx(g, s):
        ia_v, ib_v, ic_v, id_v = idx_sets[s]
        wa_v, wb_v, wc_v, wd_v = w_sets[s]
        jbase = jloc0 + g * K
        for v in range(K // 16):
            j = lanes + jnp.full((16,), jbase + v * 16, jnp.int32)
            row = lax.shift_right_logical(
                lax.shift_right_logical(j, c7) * cmag, c17)
            col = j - row * wsplat
            gx = bf16r(col.astype(jnp.float32) * STEP - 1.0)
            gy = bf16r(row.astype(jnp.float32) * STEP - 1.0)
            xs = t0 * gx + t1 * gy + t2
            ys = t3 * gx + t4 * gy + t5
            px = 0.5 * (xs + 1.0) * float(W)
            py = 0.5 * (ys + 1.0) * float(H)
            x0 = px.astype(jnp.int32)
            y0 = py.astype(jnp.int32)
            x1 = x0 + ione
            y1 = y0 + ione
            x0c = jnp.minimum(jnp.maximum(x0, izero), iwmax)
            x1c = jnp.minimum(jnp.maximum(x1, izero), iwmax)
            y0c = jnp.minimum(jnp.maximum(y0, izero), ihmax)
            y1c = jnp.minimum(jnp.maximum(y1, izero), ihmax)
            x0f = x0c.astype(jnp.float32)
            x1f = x1c.astype(jnp.float32)
            y0f = y0c.astype(jnp.float32)
            y1f = y1c.astype(jnp.float32)
            sl = pl.ds(v * 16, 16)
            ia_v[sl] = bvec + y0c * wsplat + x0c
            ib_v[sl] = bvec + y1c * wsplat + x0c
            ic_v[sl] = bvec + y0c * wsplat + x1c
            id_v[sl] = bvec + y1c * wsplat + x1c
            wa_v[v, :] = (x1f - px) * (y1f - py)
            wb_v[v, :] = (x1f - px) * (py - y0f)
            wc_v[v, :] = (px - x0f) * (y1f - py)
            wd_v[v, :] = (px - x0f) * (py - y0f)

    def fire(s):
        ia_v, ib_v, ic_v, id_v = idx_sets[s]
        ra_v, rb_v, rc_v, rd_v = row_sets[s]
        pltpu.async_copy(x_hbm.at[ia_v], ra_v, gsems[s])
        pltpu.async_copy(x_hbm.at[ib_v], rb_v, gsems[s])
        pltpu.async_copy(x_hbm.at[ic_v], rc_v, gsems[s])
        pltpu.async_copy(x_hbm.at[id_v], rd_v, gsems[s])

    def drain_gathers(s):
        ia_v, ib_v, ic_v, id_v = idx_sets[s]
        ra_v, rb_v, rc_v, rd_v = row_sets[s]
        pltpu.make_async_copy(x_hbm.at[ia_v], ra_v, gsems[s]).wait()
        pltpu.make_async_copy(x_hbm.at[ib_v], rb_v, gsems[s]).wait()
        pltpu.make_async_copy(x_hbm.at[ic_v], rc_v, gsems[s]).wait()
        pltpu.make_async_copy(x_hbm.at[id_v], rd_v, gsems[s]).wait()

    def drain_out(s):
        pltpu.make_async_copy(
            out_sets[s], out_hbm.at[pl.ds(row0, K)], osems[s]).wait()

    def combine(g, s):
        wa_v, wb_v, wc_v, wd_v = w_sets[s]
        ra_v, rb_v, rc_v, rd_v = row_sets[s]
        out_v = out_sets[s]

        def grp(gg, carry2):
            wa16 = wa_v[gg, :]
            wb16 = wb_v[gg, :]
            wc16 = wc_v[gg, :]
            wd16 = wd_v[gg, :]
            for r in range(16):
                rowi = gg * 16 + r
                wav = jnp.full((16,), wa16[r], jnp.float32)
                wbv = jnp.full((16,), wb16[r], jnp.float32)
                wcv = jnp.full((16,), wc16[r], jnp.float32)
                wdv = jnp.full((16,), wd16[r], jnp.float32)
                for cc in range(C // 16):
                    slc = pl.ds(cc * 16, 16)
                    o = (wav * ra_v[rowi, slc] + wbv * rb_v[rowi, slc]
                         + wcv * rc_v[rowi, slc] + wdv * rd_v[rowi, slc])
                    out_v[rowi, slc] = o
            return carry2

        lax.fori_loop(0, K // 16, grp, 0)
        pltpu.async_copy(out_v, out_hbm.at[pl.ds(row0 + g * K, K)], osems[s])

    fill_idx(0, 0)
    fire(0)

    def pair(gp, carry):
        for par in (0, 1):
            g = gp * 2 + par
            nxt = 1 - par

            @pl.when(g + 1 < NCHUNK)
            def _prefetch():
                @pl.when(g >= 1)
                def _reclaim():
                    drain_out(nxt)

                fill_idx(g + 1, nxt)
                fire(nxt)

            drain_gathers(par)
            combine(g, par)
        return carry

    lax.fori_loop(0, NCHUNK // 2, pair, 0)
    drain_out(0)
    drain_out(1)


@functools.partial(jax.jit, donate_argnums=())
def _sc_interp(xflat, tbcast):
    mesh = plsc.VectorSubcoreMesh(core_axis_name="c", subcore_axis_name="s")
    idx_t = pltpu.VMEM((K,), jnp.int32)
    w_t = pltpu.VMEM((K // 16, 16), jnp.float32)
    rows_t = pltpu.VMEM((K, C), jnp.float32)
    f = functools.partial(
        pl.kernel,
        out_type=jax.ShapeDtypeStruct((N, C), jnp.float32),
        mesh=mesh,
        compiler_params=pltpu.CompilerParams(use_tc_tiling_on_sc=False),
        scratch_types=(
            [pltpu.VMEM((6, 16), jnp.float32)]
            + [idx_t] * 8 + [w_t] * 8 + [rows_t] * 8
            + [pltpu.SemaphoreType.DMA] * 4
        ),
    )(_interp_body)
    return f(xflat, tbcast)


def kernel(X, transformation):
    xflat = X.reshape(N, C)
    tb = jnp.broadcast_to(
        transformation.astype(jnp.float32)[:, :, None], (B, 6, 16))
    out = _sc_interp(xflat, tb)
    return out.reshape(B, H, W, C)


# speedup vs baseline: 1.0633x; 1.0633x over previous
import functools

import jax
import jax.numpy as jnp
import numpy as np
from jax import lax
from jax.experimental import pallas as pl
from jax.experimental.pallas import tpu as pltpu
from jax.experimental.pallas import tpu_sc as plsc

B, H, W, C = 4, 384, 384, 96
CP = 128
HW = H * W
N = B * HW
NW = 32
ROWS_PER_TILE = N // NW
K = 64
NCHUNK = ROWS_PER_TILE // K
TILES_PER_BATCH = NW // B
STEP = float(np.float32(2.0) / np.float32(W - 1))


def _sdiv_w(j):
    return ((j >> 7) * 43691) >> 17


def _interp_body(x_hbm, t_hbm, out_hbm, t_v,
                 ia0, ib0, ic0, id0, ia1, ib1, ic1, id1,
                 wa0, wb0, wc0, wd0, wa1, wb1, wc1, wd1,
                 ra0, rb0, rc0, rd0, ra1, rb1, rc1, rd1,
                 out0, out1, gsem0, gsem1, osem0, osem1):
    wid = lax.axis_index("s") * 2 + lax.axis_index("c")
    b = wid // TILES_PER_BATCH
    pltpu.sync_copy(t_hbm.at[b], t_v)
    vsplit = jnp.full((16,), 65537.0, jnp.float32)

    def bf16r(x):
        p = x * vsplit
        q = p - x
        return p - q

    t0 = bf16r(t_v[0, :])
    t1 = bf16r(t_v[1, :])
    t2 = bf16r(t_v[2, :])
    t3 = bf16r(t_v[3, :])
    t4 = bf16r(t_v[4, :])
    t5 = bf16r(t_v[5, :])
    jloc0 = (wid % TILES_PER_BATCH) * ROWS_PER_TILE
    bbase = b * HW
    lanes = lax.iota(jnp.int32, 16)

    izero = jnp.zeros((16,), jnp.int32)
    iwmax = jnp.full((16,), W - 1, jnp.int32)
    ihmax = jnp.full((16,), H - 1, jnp.int32)
    ione = jnp.full((16,), 1, jnp.int32)
    wsplat = jnp.full((16,), W, jnp.int32)
    c7 = jnp.full((16,), 7, jnp.int32)
    c17 = jnp.full((16,), 17, jnp.int32)
    cmag = jnp.full((16,), 43691, jnp.int32)
    bvec = jnp.full((16,), bbase, jnp.int32)

    idx_sets = ((ia0, ib0, ic0, id0), (ia1, ib1, ic1, id1))
    w_sets = ((wa0, wb0, wc0, wd0), (wa1, wb1, wc1, wd1))
    row_sets = ((ra0, rb0, rc0, rd0), (ra1, rb1, rc1, rd1))
    out_sets = (out0, out1)
    gsems = (gsem0, gsem1)
    osems = (osem0, osem1)

    def fill_idx(g, s):
        ia_v, ib_v, ic_v, id_v = idx_sets[s]
        wa_v, wb_v, wc_v, wd_v = w_sets[s]
        jbase = jloc0 + g * K
        for v in range(K // 16):
            j = lanes + jnp.full((16,), jbase + v * 16, jnp.int32)
            row = lax.shift_right_logical(
                lax.shift_right_logical(j, c7) * cmag, c17)
            col = j - row * wsplat
            gx = bf16r(col.astype(jnp.float32) * STEP - 1.0)
            gy = bf16r(row.astype(jnp.float32) * STEP - 1.0)
            xs = t0 * gx + t1 * gy + t2
            ys = t3 * gx + t4 * gy + t5
            px = 0.5 * (xs + 1.0) * float(W)
            py = 0.5 * (ys + 1.0) * float(H)
            x0 = px.astype(jnp.int32)
            y0 = py.astype(jnp.int32)
            x1 = x0 + ione
            y1 = y0 + ione
            x0c = jnp.minimum(jnp.maximum(x0, izero), iwmax)
            x1c = jnp.minimum(jnp.maximum(x1, izero), iwmax)
            y0c = jnp.minimum(jnp.maximum(y0, izero), ihmax)
            y1c = jnp.minimum(jnp.maximum(y1, izero), ihmax)
            x0f = x0c.astype(jnp.float32)
            x1f = x1c.astype(jnp.float32)
            y0f = y0c.astype(jnp.float32)
            y1f = y1c.astype(jnp.float32)
            sl = pl.ds(v * 16, 16)
            ia_v[sl] = bvec + y0c * wsplat + x0c
            ib_v[sl] = bvec + y1c * wsplat + x0c
            ic_v[sl] = bvec + y0c * wsplat + x1c
            id_v[sl] = bvec + y1c * wsplat + x1c
            wa_v[v, :] = (x1f - px) * (y1f - py)
            wb_v[v, :] = (x1f - px) * (py - y0f)
            wc_v[v, :] = (px - x0f) * (y1f - py)
            wd_v[v, :] = (px - x0f) * (py - y0f)

    def fire(s):
        ia_v, ib_v, ic_v, id_v = idx_sets[s]
        ra_v, rb_v, rc_v, rd_v = row_sets[s]
        pltpu.async_copy(x_hbm.at[ia_v], ra_v, gsems[s])
        pltpu.async_copy(x_hbm.at[ib_v], rb_v, gsems[s])
        pltpu.async_copy(x_hbm.at[ic_v], rc_v, gsems[s])
        pltpu.async_copy(x_hbm.at[id_v], rd_v, gsems[s])

    def drain_gathers(s):
        ia_v, ib_v, ic_v, id_v = idx_sets[s]
        ra_v, rb_v, rc_v, rd_v = row_sets[s]
        pltpu.make_async_copy(x_hbm.at[ia_v], ra_v, gsems[s]).wait()
        pltpu.make_async_copy(x_hbm.at[ib_v], rb_v, gsems[s]).wait()
        pltpu.make_async_copy(x_hbm.at[ic_v], rc_v, gsems[s]).wait()
        pltpu.make_async_copy(x_hbm.at[id_v], rd_v, gsems[s]).wait()

    def out_slice(g):
        jbase = jloc0 + g * K
        y = _sdiv_w(jbase)
        xs0 = jbase - y * W
        return out_hbm.at[b, y, pl.ds(xs0, K)]

    def drain_out(s):
        pltpu.make_async_copy(out_sets[s], out_slice(0), osems[s]).wait()

    def combine(g, s):
        wa_v, wb_v, wc_v, wd_v = w_sets[s]
        ra_v, rb_v, rc_v, rd_v = row_sets[s]
        out_v = out_sets[s]

        def grp(gg, carry2):
            wa16 = wa_v[gg, :]
            wb16 = wb_v[gg, :]
            wc16 = wc_v[gg, :]
            wd16 = wd_v[gg, :]
            for r in range(16):
                rowi = gg * 16 + r
                wav = jnp.full((16,), wa16[r], jnp.float32)
                wbv = jnp.full((16,), wb16[r], jnp.float32)
                wcv = jnp.full((16,), wc16[r], jnp.float32)
                wdv = jnp.full((16,), wd16[r], jnp.float32)
                for cc in range(C // 16):
                    slc = pl.ds(cc * 16, 16)
                    o = (wav * ra_v[rowi, slc] + wbv * rb_v[rowi, slc]
                         + wcv * rc_v[rowi, slc] + wdv * rd_v[rowi, slc])
                    out_v[rowi, slc] = o
            return carry2

        lax.fori_loop(0, K // 16, grp, 0)
        pltpu.async_copy(out_v, out_slice(g), osems[s])

    fill_idx(0, 0)
    fire(0)

    def pair(gp, carry):
        for par in (0, 1):
            g = gp * 2 + par
            nxt = 1 - par

            @pl.when(g + 1 < NCHUNK)
            def _prefetch():
                fill_idx(g + 1, nxt)
                fire(nxt)

            drain_gathers(par)

            @pl.when(g >= 2)
            def _reclaim():
                drain_out(par)

            combine(g, par)
        return carry

    lax.fori_loop(0, NCHUNK // 2, pair, 0)
    drain_out(0)
    drain_out(1)


@functools.partial(jax.jit, donate_argnums=())
def _sc_interp(xpad, tbcast):
    mesh = plsc.VectorSubcoreMesh(core_axis_name="c", subcore_axis_name="s")
    idx_t = pltpu.VMEM((K,), jnp.int32)
    w_t = pltpu.VMEM((K // 16, 16), jnp.float32)
    rows_t = pltpu.VMEM((K, CP), jnp.float32)
    f = functools.partial(
        pl.kernel,
        out_type=jax.ShapeDtypeStruct((B, H, W, C), jnp.float32),
        mesh=mesh,
        scratch_types=(
            [pltpu.VMEM((6, 16), jnp.float32)]
            + [idx_t] * 8 + [w_t] * 8 + [rows_t] * 8
            + [pltpu.VMEM((K, C), jnp.float32)] * 2
            + [pltpu.SemaphoreType.DMA] * 4
        ),
    )(_interp_body)
    return f(xpad, tbcast)


def kernel(X, transformation):
    xpad = jnp.pad(X.reshape(N, C), ((0, 0), (0, CP - C)))
    tb = jnp.broadcast_to(
        transformation.astype(jnp.float32)[:, :, None], (B, 6, 16))
    return _sc_interp(xpad, tb)


# speedup vs baseline: 1.1351x; 1.0675x over previous
import functools

import jax
import jax.numpy as jnp
import numpy as np
from jax import lax
from jax.experimental import pallas as pl
from jax.experimental.pallas import tpu as pltpu
from jax.experimental.pallas import tpu_sc as plsc

B, H, W, C = 4, 384, 384, 96
CP = 128
HW = H * W
N = B * HW
NW = 32
ROWS_PER_TILE = N // NW
K = 96
NCHUNK = ROWS_PER_TILE // K
TILES_PER_BATCH = NW // B
STEP = float(np.float32(2.0) / np.float32(W - 1))


def _sdiv_w(j):
    return ((j >> 7) * 43691) >> 17


def _interp_body(x_hbm, t_hbm, out_hbm, t_v,
                 ia0, ib0, ic0, id0, ia1, ib1, ic1, id1,
                 wa0, wb0, wc0, wd0, wa1, wb1, wc1, wd1,
                 ra0, rb0, rc0, rd0, ra1, rb1, rc1, rd1,
                 outb, gsem0, gsem1, osem):
    wid = lax.axis_index("s") * 2 + lax.axis_index("c")
    b = wid // TILES_PER_BATCH
    pltpu.sync_copy(t_hbm.at[b], t_v)
    vsplit = jnp.full((16,), 65537.0, jnp.float32)

    def bf16r(x):
        p = x * vsplit
        q = p - x
        return p - q

    t0 = bf16r(t_v[0, :])
    t1 = bf16r(t_v[1, :])
    t2 = bf16r(t_v[2, :])
    t3 = bf16r(t_v[3, :])
    t4 = bf16r(t_v[4, :])
    t5 = bf16r(t_v[5, :])
    jloc0 = (wid % TILES_PER_BATCH) * ROWS_PER_TILE
    bbase = b * HW
    lanes = lax.iota(jnp.int32, 16)

    izero = jnp.zeros((16,), jnp.int32)
    iwmax = jnp.full((16,), W - 1, jnp.int32)
    ihmax = jnp.full((16,), H - 1, jnp.int32)
    ione = jnp.full((16,), 1, jnp.int32)
    wsplat = jnp.full((16,), W, jnp.int32)
    c7 = jnp.full((16,), 7, jnp.int32)
    c17 = jnp.full((16,), 17, jnp.int32)
    cmag = jnp.full((16,), 43691, jnp.int32)
    bvec = jnp.full((16,), bbase, jnp.int32)

    idx_sets = ((ia0, ib0, ic0, id0), (ia1, ib1, ic1, id1))
    w_sets = ((wa0, wb0, wc0, wd0), (wa1, wb1, wc1, wd1))
    row_sets = ((ra0, rb0, rc0, rd0), (ra1, rb1, rc1, rd1))
    gsems = (gsem0, gsem1)

    def fill_idx(g, s):
        ia_v, ib_v, ic_v, id_v = idx_sets[s]
        wa_v, wb_v, wc_v, wd_v = w_sets[s]
        jbase = jloc0 + g * K
        for v in range(K // 16):
            j = lanes + jnp.full((16,), jbase + v * 16, jnp.int32)
            row = lax.shift_right_logical(
                lax.shift_right_logical(j, c7) * cmag, c17)
            col = j - row * wsplat
            gx = bf16r(col.astype(jnp.float32) * STEP - 1.0)
            gy = bf16r(row.astype(jnp.float32) * STEP - 1.0)
            xs = t0 * gx + t1 * gy + t2
            ys = t3 * gx + t4 * gy + t5
            px = 0.5 * (xs + 1.0) * float(W)
            py = 0.5 * (ys + 1.0) * float(H)
            x0 = px.astype(jnp.int32)
            y0 = py.astype(jnp.int32)
            x1 = x0 + ione
            y1 = y0 + ione
            x0c = jnp.minimum(jnp.maximum(x0, izero), iwmax)
            x1c = jnp.minimum(jnp.maximum(x1, izero), iwmax)
            y0c = jnp.minimum(jnp.maximum(y0, izero), ihmax)
            y1c = jnp.minimum(jnp.maximum(y1, izero), ihmax)
            x0f = x0c.astype(jnp.float32)
            x1f = x1c.astype(jnp.float32)
            y0f = y0c.astype(jnp.float32)
            y1f = y1c.astype(jnp.float32)
            sl = pl.ds(v * 16, 16)
            ia_v[sl] = bvec + y0c * wsplat + x0c
            ib_v[sl] = bvec + y1c * wsplat + x0c
            ic_v[sl] = bvec + y0c * wsplat + x1c
            id_v[sl] = bvec + y1c * wsplat + x1c
            wa_v[sl] = (x1f - px) * (y1f - py)
            wb_v[sl] = (x1f - px) * (py - y0f)
            wc_v[sl] = (px - x0f) * (y1f - py)
            wd_v[sl] = (px - x0f) * (py - y0f)

    def fire(s):
        ia_v, ib_v, ic_v, id_v = idx_sets[s]
        ra_v, rb_v, rc_v, rd_v = row_sets[s]
        pltpu.async_copy(x_hbm.at[ia_v], ra_v, gsems[s])
        pltpu.async_copy(x_hbm.at[ib_v], rb_v, gsems[s])
        pltpu.async_copy(x_hbm.at[ic_v], rc_v, gsems[s])
        pltpu.async_copy(x_hbm.at[id_v], rd_v, gsems[s])

    def drain_gathers(s):
        ia_v, ib_v, ic_v, id_v = idx_sets[s]
        ra_v, rb_v, rc_v, rd_v = row_sets[s]
        pltpu.make_async_copy(x_hbm.at[ia_v], ra_v, gsems[s]).wait()
        pltpu.make_async_copy(x_hbm.at[ib_v], rb_v, gsems[s]).wait()
        pltpu.make_async_copy(x_hbm.at[ic_v], rc_v, gsems[s]).wait()
        pltpu.make_async_copy(x_hbm.at[id_v], rd_v, gsems[s]).wait()

    def out_slice(g):
        jbase = jloc0 + g * K
        y = _sdiv_w(jbase)
        xs0 = jbase - y * W
        return out_hbm.at[b, y, pl.ds(xs0, K)]

    def drain_out():
        pltpu.make_async_copy(outb, out_slice(0), osem).wait()

    def combine(g, s):
        wa_v, wb_v, wc_v, wd_v = w_sets[s]
        ra_v, rb_v, rc_v, rd_v = row_sets[s]
        out_v = outb

        def grp(gg, carry2):
            gsl = pl.ds(gg * 16, 16)
            wa16 = wa_v[gsl]
            wb16 = wb_v[gsl]
            wc16 = wc_v[gsl]
            wd16 = wd_v[gsl]
            for r in range(16):
                rowi = gg * 16 + r
                wav = jnp.full((16,), wa16[r], jnp.float32)
                wbv = jnp.full((16,), wb16[r], jnp.float32)
                wcv = jnp.full((16,), wc16[r], jnp.float32)
                wdv = jnp.full((16,), wd16[r], jnp.float32)
                for cc in range(C // 16):
                    slc = pl.ds(cc * 16, 16)
                    o = (wav * ra_v[rowi, slc] + wbv * rb_v[rowi, slc]
                         + wcv * rc_v[rowi, slc] + wdv * rd_v[rowi, slc])
                    out_v[rowi, slc] = o
            return carry2

        lax.fori_loop(0, K // 16, grp, 0)
        pltpu.async_copy(outb, out_slice(g), osem)

    fill_idx(0, 0)
    fire(0)

    def pair(gp, carry):
        for par in (0, 1):
            g = gp * 2 + par
            nxt = 1 - par

            @pl.when(g + 1 < NCHUNK)
            def _prefetch():
                fill_idx(g + 1, nxt)
                fire(nxt)

            drain_gathers(par)

            @pl.when(g >= 1)
            def _reclaim():
                drain_out()

            combine(g, par)
        return carry

    lax.fori_loop(0, NCHUNK // 2, pair, 0)
    drain_out()


def _sc_interp(xpad, tbcast):
    mesh = plsc.VectorSubcoreMesh(core_axis_name="c", subcore_axis_name="s")
    idx_t = pltpu.VMEM((K,), jnp.int32)
    w_t = pltpu.VMEM((K,), jnp.float32)
    rows_t = pltpu.VMEM((K, CP), jnp.float32)
    f = functools.partial(
        pl.kernel,
        out_type=jax.ShapeDtypeStruct((B, H, W, C), jnp.float32),
        mesh=mesh,
        scratch_types=(
            [pltpu.VMEM((6, 16), jnp.float32)]
            + [idx_t] * 8 + [w_t] * 8 + [rows_t] * 8
            + [pltpu.VMEM((K, C), jnp.float32)]
            + [pltpu.SemaphoreType.DMA] * 3
        ),
    )(_interp_body)
    return f(xpad, tbcast)


@jax.jit
def kernel(X, transformation):
    xpad = jnp.pad(X.reshape(N, C), ((0, 0), (0, CP - C)))
    tb = jnp.broadcast_to(
        transformation.astype(jnp.float32)[:, :, None], (B, 6, 16))
    return _sc_interp(xpad, tb)
